# Initial kernel scaffold; baseline (speedup 1.0000x reference)
import jax
import jax.numpy as jnp
from jax.experimental import pallas as pl


def kernel(input_vectors, partition_indices, som_weights, meta_weights):
    raise NotImplementedError("write your pallas kernel here")


# speedup vs baseline: 1.1020x; 1.1020x over previous
import functools

import jax
import jax.numpy as jnp
from jax import lax
from jax.experimental import pallas as pl
from jax.experimental.pallas import tpu as pltpu
from jax.experimental.pallas import tpu_sc as plsc


def _make_som_body(g_per_som, sg):
    def body(x_ref, part_ref, wt_ref, out_ref):
        x = x_ref[...]
        wt = wt_ref[...]
        x2 = jnp.sum(x * x, axis=1, keepdims=True)
        w2 = jnp.sum(wt * wt, axis=0, keepdims=True)
        xw = lax.dot_general(
            x, wt,
            (((1,), (0,)), ((), ())),
            preferred_element_type=jnp.float32,
        )
        scores = x2 - 2.0 * xw + w2
        col = lax.broadcasted_iota(jnp.int32, scores.shape, 1)
        mask = (col // g_per_som) != part_ref[...]
        scores = jnp.where(mask, jnp.inf, scores)
        mn = jnp.min(scores, axis=1, keepdims=True)
        out_ref[...] = jnp.min(
            jnp.where(scores <= mn, col, sg), axis=1, keepdims=True
        )
    return body


def _make_meta_body(ncols, my):
    def body(x_ref, wt_ref, w2_ref, idx_ref, cx_ref, cy_ref):
        x = x_ref[...]
        wt = wt_ref[...]
        x2 = jnp.sum(x * x, axis=1, keepdims=True)
        xw = lax.dot_general(
            x, wt,
            (((1,), (0,)), ((), ())),
            preferred_element_type=jnp.float32,
        )
        scores = x2 - 2.0 * xw + w2_ref[...]
        col = lax.broadcasted_iota(jnp.int32, scores.shape, 1)
        scores = jnp.where(col >= ncols, jnp.inf, scores)
        mn = jnp.min(scores, axis=1, keepdims=True)
        idx = jnp.min(
            jnp.where(scores <= mn, col, ncols), axis=1, keepdims=True
        )
        idx_ref[...] = idx
        cx_ref[...] = idx // my
        cy_ref[...] = idx % my
    return body


def _tc_som_argmin(x, part2d, wt, g_per_som, bn=256):
    n, d = x.shape
    sg = wt.shape[1]
    return pl.pallas_call(
        _make_som_body(g_per_som, sg),
        grid=(n // bn,),
        in_specs=[
            pl.BlockSpec((bn, d), lambda i: (i, 0)),
            pl.BlockSpec((bn, 1), lambda i: (i, 0)),
            pl.BlockSpec((d, sg), lambda i: (0, 0)),
        ],
        out_specs=pl.BlockSpec((bn, 1), lambda i: (i, 0)),
        out_shape=jax.ShapeDtypeStruct((n, 1), jnp.int32),
    )(x, part2d, wt)


def _tc_meta_argmin(x, wt, w2, ncols, my, bn=256):
    n = x.shape[0]
    d, mp = wt.shape
    o = jax.ShapeDtypeStruct((n, 1), jnp.int32)
    return pl.pallas_call(
        _make_meta_body(ncols, my),
        grid=(n // bn,),
        in_specs=[
            pl.BlockSpec((bn, d), lambda i: (i, 0)),
            pl.BlockSpec((d, mp), lambda i: (0, 0)),
            pl.BlockSpec((1, mp), lambda i: (0, 0)),
        ],
        out_specs=[pl.BlockSpec((bn, 1), lambda i: (i, 0))] * 3,
        out_shape=[o, o, o],
    )(x, wt, w2)


def _sc_gather(table, idx):
    info = plsc.get_sparse_core_info()
    nw = info.num_cores * info.num_subcores
    b, d = idx.shape[0], table.shape[1]
    bw = b // nw
    mesh = plsc.VectorSubcoreMesh(core_axis_name="c", subcore_axis_name="s")

    @functools.partial(
        pl.kernel,
        mesh=mesh,
        out_type=jax.ShapeDtypeStruct((b, d), table.dtype),
        scratch_types=[
            pltpu.VMEM((bw,), jnp.int32),
            pltpu.VMEM((bw, d), table.dtype),
            pltpu.SemaphoreType.DMA,
        ],
    )
    def k(table_hbm, idx_hbm, out_hbm, idx_v, rows_v, sem):
        wid = lax.axis_index("s") * info.num_cores + lax.axis_index("c")
        base = wid * bw
        pltpu.sync_copy(idx_hbm.at[pl.ds(base, bw)], idx_v)
        pltpu.async_copy(table_hbm.at[idx_v], rows_v, sem).wait()
        pltpu.sync_copy(rows_v, out_hbm.at[pl.ds(base, bw)])

    return k(table, idx)


@jax.jit
def kernel(input_vectors, partition_indices, som_weights, meta_weights):
    n, d = input_vectors.shape
    s = som_weights.shape[0]
    g = som_weights.shape[1] * som_weights.shape[2]
    all_w = som_weights.reshape(s * g, d)
    all_w128 = jnp.pad(all_w, ((0, 0), (0, 128 - d)))

    mx, my = meta_weights.shape[0], meta_weights.shape[1]
    m = mx * my
    m_pad = ((m + 127) // 128) * 128
    mflat = meta_weights.reshape(m, d)
    mflat128 = jnp.pad(mflat, ((0, m_pad - m), (0, 128 - d)))

    m2 = jnp.pad(jnp.sum(mflat * mflat, axis=1), (0, m_pad - m))[None, :]
    mflat_t = jnp.pad(mflat, ((0, m_pad - m), (0, 0))).T

    bmu_flat = _tc_som_argmin(
        input_vectors, partition_indices.reshape(n, 1), all_w.T, g
    ).reshape(n)
    sel128 = _sc_gather(all_w128, bmu_flat)
    meta_idx, cx, cy = _tc_meta_argmin(sel128[:, :d], mflat_t, m2, m, my)
    meta_vec = _sc_gather(mflat128, meta_idx.reshape(n))[:, :d]
    coords = jnp.concatenate([cx, cy], axis=1)
    return coords, meta_vec
